# baseline (speedup 1.0000x reference)
import functools

import jax
import jax.numpy as jnp
from jax import lax
from jax.experimental import pallas as pl
from jax.experimental.pallas import tpu as pltpu
from jax.experimental.pallas import tpu_sc as plsc

NUM_NODES = 1000000
EMB = 64
BATCH = 16384

_NC = 2
_NS = 16
_NW = _NC * _NS
_BPW = BATCH // _NW
_CHUNK = 128
_NCHUNK = _BPW // _CHUNK


def _make_lookup():
  mesh = plsc.VectorSubcoreMesh(core_axis_name="c", subcore_axis_name="s")

  @functools.partial(
      pl.kernel,
      mesh=mesh,
      out_type=jax.ShapeDtypeStruct((BATCH, EMB), jnp.float32),
      compiler_params=pltpu.CompilerParams(use_tc_tiling_on_sc=False),
      scratch_types=[
          pltpu.VMEM((_NCHUNK, _CHUNK), jnp.int32),
          pltpu.VMEM((_BPW, EMB), jnp.float32),
          pltpu.SemaphoreType.DMA,
      ],
  )
  def emb_lookup(idx_hbm, table_hbm, out_hbm, idx_v, rows_v, sem):
    wid = lax.axis_index("s") * _NC + lax.axis_index("c")
    base = wid * _BPW
    pltpu.sync_copy(idx_hbm.at[wid], idx_v)
    copies = []
    for j in range(_NCHUNK):
      copies.append(
          pltpu.async_copy(
              table_hbm.at[idx_v.at[j]],
              rows_v.at[pl.ds(j * _CHUNK, _CHUNK)],
              sem,
          ))
    for c in copies:
      c.wait()
    pltpu.sync_copy(rows_v, out_hbm.at[pl.ds(base, _BPW)])

  return emb_lookup


_emb_lookup = _make_lookup()


def kernel(idx, emb_weight):
  idx3 = idx.astype(jnp.int32).reshape(_NW, _NCHUNK, _CHUNK)
  return _emb_lookup(idx3, emb_weight)


# speedup vs baseline: 2.6011x; 2.6011x over previous
import functools

import jax
import jax.numpy as jnp
from jax import lax
from jax.experimental import pallas as pl
from jax.experimental.pallas import tpu as pltpu
from jax.experimental.pallas import tpu_sc as plsc

NUM_NODES = 1000000
EMB = 64
BATCH = 16384

_NC = 2
_NS = 16
_NW = _NC * _NS
_BPW = BATCH // _NW
_NBUF = 4


def _make_lookup():
  mesh = plsc.VectorSubcoreMesh(core_axis_name="c", subcore_axis_name="s")

  @functools.partial(
      pl.kernel,
      mesh=mesh,
      out_type=jax.ShapeDtypeStruct((EMB, BATCH), jnp.float32),
      compiler_params=pltpu.CompilerParams(
          use_tc_tiling_on_sc=True, needs_layout_passes=False),
      scratch_types=[
          pltpu.VMEM((_BPW,), jnp.int32),
          pltpu.VMEM((EMB, 128), jnp.float32),
          pltpu.VMEM((EMB, 128), jnp.float32),
          pltpu.VMEM((EMB, 128), jnp.float32),
          pltpu.VMEM((EMB, 128), jnp.float32),
          pltpu.VMEM((EMB, _BPW), jnp.float32),
          pltpu.SemaphoreType.DMA((_NBUF,)),
      ],
  )
  def emb_lookup(idx_hbm, tbl_hbm, out_hbm, idx_v, s0, s1, s2, s3, out_v,
                 sems):
    slabs = [s0, s1, s2, s3]
    wid = lax.axis_index("s") * _NC + lax.axis_index("c")
    base = wid * _BPW
    pltpu.sync_copy(idx_hbm.at[pl.ds(base, _BPW)], idx_v)

    iota = lax.iota(jnp.int32, 16)
    row_idx = [16 * k + iota for k in range(4)]

    def get_idx(j):
      grp = pl.multiple_of((j >> 4) << 4, 16)
      v = idx_v[pl.ds(grp, 16)]
      return jnp.sum(jnp.where(iota == (j & 15), v, 0))

    def issue(j, b):
      i = get_idx(j)
      n = pl.multiple_of((i >> 7) << 7, 128)
      pltpu.async_copy(tbl_hbm.at[:, pl.ds(n, 128)], slabs[b], sems.at[b])

    def wait(j, b):
      pltpu.make_async_copy(
          tbl_hbm.at[:, pl.ds(0, 128)], slabs[b], sems.at[b]).wait()

    def extract(j, b):
      i = get_idx(j)
      lane = jnp.full((16,), i & 127, jnp.int32)
      col = jnp.full((16,), j, jnp.int32)
      for k in range(4):
        v = plsc.load_gather(slabs[b], [row_idx[k], lane])
        plsc.store_scatter(out_v, [row_idx[k], col], v)

    for b in range(_NBUF):
      issue(b, b)

    def body(t, _):
      for b in range(_NBUF):
        j = t * _NBUF + b
        wait(j, b)
        extract(j, b)
        nxt = j + _NBUF

        @pl.when(nxt < _BPW)
        def _():
          issue(nxt, b)

      return ()

    lax.fori_loop(0, _BPW // _NBUF, body, (), unroll=False)
    pltpu.sync_copy(out_v, out_hbm.at[:, pl.ds(base, _BPW)])

  return emb_lookup


_emb_lookup = _make_lookup()


def kernel(idx, emb_weight):
  out_t = _emb_lookup(idx.astype(jnp.int32), emb_weight.T)
  return out_t.T


# speedup vs baseline: 3.0175x; 1.1601x over previous
import functools

import jax
import jax.numpy as jnp
from jax import lax
from jax.experimental import pallas as pl
from jax.experimental.pallas import tpu as pltpu
from jax.experimental.pallas import tpu_sc as plsc

NUM_NODES = 1000000
EMB = 64
BATCH = 16384

_NC = 2
_NS = 16
_NW = _NC * _NS
_BPW = BATCH // _NW
_NBUF = 8


def _make_lookup():
  mesh = plsc.VectorSubcoreMesh(core_axis_name="c", subcore_axis_name="s")

  @functools.partial(
      pl.kernel,
      mesh=mesh,
      out_type=jax.ShapeDtypeStruct((EMB, BATCH), jnp.float32),
      compiler_params=pltpu.CompilerParams(
          use_tc_tiling_on_sc=True, needs_layout_passes=False),
      scratch_types=[
          pltpu.VMEM((_BPW,), jnp.int32),
          pltpu.VMEM((EMB, 128), jnp.float32),
          pltpu.VMEM((EMB, 128), jnp.float32),
          pltpu.VMEM((EMB, 128), jnp.float32),
          pltpu.VMEM((EMB, 128), jnp.float32),
          pltpu.VMEM((EMB, 128), jnp.float32),
          pltpu.VMEM((EMB, 128), jnp.float32),
          pltpu.VMEM((EMB, 128), jnp.float32),
          pltpu.VMEM((EMB, 128), jnp.float32),
          pltpu.VMEM((EMB, _BPW), jnp.float32),
          pltpu.SemaphoreType.DMA((_NBUF,)),
      ],
  )
  def emb_lookup(idx_hbm, tbl_hbm, out_hbm, idx_v, s0, s1, s2, s3, s4, s5,
                 s6, s7, out_v, sems):
    slabs = [s0, s1, s2, s3, s4, s5, s6, s7]
    wid = lax.axis_index("s") * _NC + lax.axis_index("c")
    base = wid * _BPW
    pltpu.sync_copy(idx_hbm.at[pl.ds(base, _BPW)], idx_v)

    iota = lax.iota(jnp.int32, 16)
    row_idx = [16 * k + iota for k in range(4)]

    def get_idx(j):
      grp = pl.multiple_of((j >> 4) << 4, 16)
      v = idx_v[pl.ds(grp, 16)]
      return jnp.sum(jnp.where(iota == (j & 15), v, 0))

    def issue(j, b):
      i = get_idx(j)
      n = pl.multiple_of((i >> 7) << 7, 128)
      pltpu.async_copy(tbl_hbm.at[:, pl.ds(n, 128)], slabs[b], sems.at[b])

    def wait(j, b):
      pltpu.make_async_copy(
          tbl_hbm.at[:, pl.ds(0, 128)], slabs[b], sems.at[b]).wait()

    def extract(j, b):
      i = get_idx(j)
      lane = jnp.full((16,), i & 127, jnp.int32)
      col = jnp.full((16,), j, jnp.int32)
      for k in range(4):
        v = plsc.load_gather(slabs[b], [row_idx[k], lane])
        plsc.store_scatter(out_v, [row_idx[k], col], v)

    for b in range(_NBUF):
      issue(b, b)

    def body(t, _):
      for b in range(_NBUF):
        j = t * _NBUF + b
        wait(j, b)
        extract(j, b)
        nxt = j + _NBUF

        @pl.when(nxt < _BPW)
        def _():
          issue(nxt, b)

      return ()

    lax.fori_loop(0, _BPW // _NBUF, body, (), unroll=False)
    pltpu.sync_copy(out_v, out_hbm.at[:, pl.ds(base, _BPW)])

  return emb_lookup


_emb_lookup = _make_lookup()


def kernel(idx, emb_weight):
  out_t = _emb_lookup(idx.astype(jnp.int32), emb_weight.T)
  return out_t.T
